# baseline (speedup 1.0000x reference)
import jax
import jax.numpy as jnp
from jax import lax
from jax.experimental import pallas as pl
from jax.experimental.pallas import tpu as pltpu
from jax.experimental.pallas import tpu_sc as plsc

KE = 332.0637

_NC = 2
_NS = 16
_NW = _NC * _NS
_L = 16

_E = 6400000
_T = _E // _NW
_C = 2000
_NCHUNK = _T // _C
_VPC = _C // _L


def _body(d_hbm, pa_hbm, pb_hbm, iu_hbm, iv_hbm, out_hbm,
          ta_v, tb_v, iu0, iu1, iv0, iv1, d0, d1, o0, o1,
          insem0, insem1, outsem0, outsem1):
    wid = lax.axis_index("s") * _NC + lax.axis_index("c")
    base = wid * _T

    iu_v = (iu0, iu1)
    iv_v = (iv0, iv1)
    d_v = (d0, d1)
    o_v = (o0, o1)
    insem = (insem0, insem1)
    outsem = (outsem0, outsem1)

    pltpu.sync_copy(pa_hbm, ta_v)
    pltpu.sync_copy(pb_hbm, tb_v)

    def start_in(c, b):
        off = base + c * _C
        pltpu.async_copy(iu_hbm.at[pl.ds(off, _C)], iu_v[b], insem[b])
        pltpu.async_copy(iv_hbm.at[pl.ds(off, _C)], iv_v[b], insem[b])
        pltpu.async_copy(d_hbm.at[pl.ds(off, _C)], d_v[b], insem[b])

    def wait_in(b):
        pltpu.make_async_copy(iu_hbm.at[pl.ds(0, _C)], iu_v[b],
                              insem[b]).wait()
        pltpu.make_async_copy(iv_hbm.at[pl.ds(0, _C)], iv_v[b],
                              insem[b]).wait()
        pltpu.make_async_copy(d_hbm.at[pl.ds(0, _C)], d_v[b],
                              insem[b]).wait()

    def wait_out(b):
        pltpu.make_async_copy(o_v[b], out_hbm.at[pl.ds(0, _C)],
                              outsem[b]).wait()

    start_in(0, 0)

    def pair_body(p, carry):
        for b in range(2):
            c = 2 * p + b
            if b == 0:
                start_in(c + 1, 1)
            else:
                @pl.when(p < _NCHUNK // 2 - 1)
                def _():
                    start_in(c + 1, 0)
            wait_in(b)

            @pl.when(c >= 2)
            def _():
                wait_out(b)

            ta, tb, iub, ivb, db, ob = (
                ta_v, tb_v, iu_v[b], iv_v[b], d_v[b], o_v[b])

            @plsc.parallel_loop(0, _VPC, 1, unroll=25)
            def _(k):
                s = k * _L
                iu = iub[pl.ds(s, _L)]
                iv = ivb[pl.ds(s, _L)]
                wu = plsc.bitcast(plsc.load_gather(ta, [iu >> 1]), jnp.int32)
                wv = plsc.bitcast(plsc.load_gather(tb, [iv >> 1]), jnp.int32)
                qu = plsc.bitcast((wu >> ((iu & 1) << 4)) << 16, jnp.float32)
                qv = plsc.bitcast((wv >> ((iv & 1) << 4)) << 16, jnp.float32)
                dd = db[pl.ds(s, _L)]
                r = plsc.bitcast(0x7EF311C3 - plsc.bitcast(dd, jnp.int32),
                                 jnp.float32)
                r = r * (2.0 - dd * r)
                r = r * (2.0 - dd * r)
                ob[pl.ds(s, _L)] = (KE * qu) * qv * r

            off = base + c * _C
            pltpu.async_copy(o_v[b], out_hbm.at[pl.ds(off, _C)], outsem[b])
        return carry

    lax.fori_loop(0, _NCHUNK // 2, pair_body, 0)
    wait_out(0)
    wait_out(1)


def kernel(mlmm_distances_uv, atomic_charges, mlmm_atomic_charges,
           mlmm_idx_u, mlmm_idx_v):
    pa = lax.bitcast_convert_type(
        atomic_charges.astype(jnp.bfloat16).reshape(-1, 2), jnp.float32)
    pb = lax.bitcast_convert_type(
        mlmm_atomic_charges.astype(jnp.bfloat16).reshape(-1, 2), jnp.float32)

    mesh = plsc.VectorSubcoreMesh(core_axis_name="c", subcore_axis_name="s")
    run = pl.kernel(
        _body,
        out_type=jax.ShapeDtypeStruct((_E,), jnp.float32),
        mesh=mesh,
        compiler_params=pltpu.CompilerParams(needs_layout_passes=False),
        scratch_types=[
            pltpu.VMEM((pa.shape[0],), jnp.float32),
            pltpu.VMEM((pb.shape[0],), jnp.float32),
            pltpu.VMEM((_C,), jnp.int32),
            pltpu.VMEM((_C,), jnp.int32),
            pltpu.VMEM((_C,), jnp.int32),
            pltpu.VMEM((_C,), jnp.int32),
            pltpu.VMEM((_C,), jnp.float32),
            pltpu.VMEM((_C,), jnp.float32),
            pltpu.VMEM((_C,), jnp.float32),
            pltpu.VMEM((_C,), jnp.float32),
            pltpu.SemaphoreType.DMA,
            pltpu.SemaphoreType.DMA,
            pltpu.SemaphoreType.DMA,
            pltpu.SemaphoreType.DMA,
        ],
    )
    return run(mlmm_distances_uv, pa, pb, mlmm_idx_u, mlmm_idx_v)


# speedup vs baseline: 1.2938x; 1.2938x over previous
import jax
import jax.numpy as jnp
from jax import lax
from jax.experimental import pallas as pl
from jax.experimental.pallas import tpu as pltpu
from jax.experimental.pallas import tpu_sc as plsc

KE = 332.0637

_NC = 2
_NS = 16
_NW = _NC * _NS
_L = 16

_E = 6400000
_T = _E // _NW
_C = 2000
_NCHUNK = _T // _C
_VPC = _C // _L


def _body(d_hbm, pa_hbm, pb_hbm, iu_hbm, iv_hbm, out_hbm,
          ta_v, tb_v, iu0, iu1, iv0, iv1, d0, d1, o0, o1,
          insem0, insem1, outsem0, outsem1):
    wid = lax.axis_index("s") * _NC + lax.axis_index("c")
    base = wid * _T

    iu_v = (iu0, iu1)
    iv_v = (iv0, iv1)
    d_v = (d0, d1)
    o_v = (o0, o1)
    insem = (insem0, insem1)
    outsem = (outsem0, outsem1)

    pltpu.sync_copy(pa_hbm, ta_v)
    pltpu.sync_copy(pb_hbm, tb_v)

    def start_in(c, b):
        off = base + c * _C
        pltpu.async_copy(iu_hbm.at[pl.ds(off, _C)], iu_v[b], insem[b])
        pltpu.async_copy(iv_hbm.at[pl.ds(off, _C)], iv_v[b], insem[b])
        pltpu.async_copy(d_hbm.at[pl.ds(off, _C)], d_v[b], insem[b])

    def wait_in(b):
        pltpu.make_async_copy(iu_hbm.at[pl.ds(0, _C)], iu_v[b],
                              insem[b]).wait()
        pltpu.make_async_copy(iv_hbm.at[pl.ds(0, _C)], iv_v[b],
                              insem[b]).wait()
        pltpu.make_async_copy(d_hbm.at[pl.ds(0, _C)], d_v[b],
                              insem[b]).wait()

    def wait_out(b):
        pltpu.make_async_copy(o_v[b], out_hbm.at[pl.ds(0, _C)],
                              outsem[b]).wait()

    start_in(0, 0)

    def pair_body(p, carry):
        for b in range(2):
            c = 2 * p + b
            if b == 0:
                start_in(c + 1, 1)
            else:
                @pl.when(p < _NCHUNK // 2 - 1)
                def _():
                    start_in(c + 1, 0)
            wait_in(b)

            @pl.when(c >= 2)
            def _():
                wait_out(b)

            ta, tb, iub, ivb, db, ob = (
                ta_v, tb_v, iu_v[b], iv_v[b], d_v[b], o_v[b])

            @plsc.parallel_loop(0, _VPC, 1, unroll=5)
            def _(k):
                s = k * _L
                iu = iub[pl.ds(s, _L)]
                iv = ivb[pl.ds(s, _L)]
                wu = plsc.bitcast(plsc.load_gather(ta, [iu >> 1]), jnp.int32)
                wv = plsc.bitcast(plsc.load_gather(tb, [iv >> 1]), jnp.int32)
                qu = plsc.bitcast((wu >> ((iu & 1) << 4)) << 16, jnp.float32)
                qv = plsc.bitcast((wv >> ((iv & 1) << 4)) << 16, jnp.float32)
                dd = db[pl.ds(s, _L)]
                r = plsc.bitcast(0x7EF311C3 - plsc.bitcast(dd, jnp.int32),
                                 jnp.float32)
                r = r * (2.0 - dd * r)
                r = r * (2.0 - dd * r)
                ob[pl.ds(s, _L)] = (KE * qu) * qv * r

            off = base + c * _C
            pltpu.async_copy(o_v[b], out_hbm.at[pl.ds(off, _C)], outsem[b])
        return carry

    lax.fori_loop(0, _NCHUNK // 2, pair_body, 0)
    wait_out(0)
    wait_out(1)


def kernel(mlmm_distances_uv, atomic_charges, mlmm_atomic_charges,
           mlmm_idx_u, mlmm_idx_v):
    pa = lax.bitcast_convert_type(
        atomic_charges.astype(jnp.bfloat16).reshape(-1, 2), jnp.float32)
    pb = lax.bitcast_convert_type(
        mlmm_atomic_charges.astype(jnp.bfloat16).reshape(-1, 2), jnp.float32)

    mesh = plsc.VectorSubcoreMesh(core_axis_name="c", subcore_axis_name="s")
    run = pl.kernel(
        _body,
        out_type=jax.ShapeDtypeStruct((_E,), jnp.float32),
        mesh=mesh,
        compiler_params=pltpu.CompilerParams(needs_layout_passes=False),
        scratch_types=[
            pltpu.VMEM((pa.shape[0],), jnp.float32),
            pltpu.VMEM((pb.shape[0],), jnp.float32),
            pltpu.VMEM((_C,), jnp.int32),
            pltpu.VMEM((_C,), jnp.int32),
            pltpu.VMEM((_C,), jnp.int32),
            pltpu.VMEM((_C,), jnp.int32),
            pltpu.VMEM((_C,), jnp.float32),
            pltpu.VMEM((_C,), jnp.float32),
            pltpu.VMEM((_C,), jnp.float32),
            pltpu.VMEM((_C,), jnp.float32),
            pltpu.SemaphoreType.DMA,
            pltpu.SemaphoreType.DMA,
            pltpu.SemaphoreType.DMA,
            pltpu.SemaphoreType.DMA,
        ],
    )
    return run(mlmm_distances_uv, pa, pb, mlmm_idx_u, mlmm_idx_v)


# speedup vs baseline: 1.5046x; 1.1629x over previous
import jax
import jax.numpy as jnp
from jax import lax
from jax.experimental import pallas as pl
from jax.experimental.pallas import tpu as pltpu
from jax.experimental.pallas import tpu_sc as plsc

KE = 332.0637

_NC = 2
_NS = 16
_NW = _NC * _NS
_L = 16

_E = 6400000
_T = _E // _NW
_C = 2000
_NCHUNK = _T // _C
_VPC = _C // _L

_N_TAB = 100000
_H = _N_TAB // 2
_PCHUNK = _H // _C


def _body(d_hbm, a_hbm, b_hbm, iu_hbm, iv_hbm, out_hbm,
          ta_v, tb_v, iu0, iu1, iv0, iv1, d0, d1, o0, o1,
          insem0, insem1, outsem0, outsem1):
    wid = lax.axis_index("s") * _NC + lax.axis_index("c")
    base = wid * _T

    iu_v = (iu0, iu1)
    iv_v = (iv0, iv1)
    d_v = (d0, d1)
    o_v = (o0, o1)
    insem = (insem0, insem1)
    outsem = (outsem0, outsem1)

    def pack_table(t_hbm, tab_v):
        def pack_chunk(c, carry):
            off = c * _C
            pltpu.sync_copy(t_hbm.at[pl.ds(off, _C)], d0)
            pltpu.sync_copy(t_hbm.at[pl.ds(_H + off, _C)], d1)

            @plsc.parallel_loop(0, _VPC, 1, unroll=5)
            def _(k):
                s = k * _L
                wl = plsc.bitcast(d0[pl.ds(s, _L)], jnp.int32) + 0x8000
                wh = plsc.bitcast(d1[pl.ds(s, _L)], jnp.int32) + 0x8000
                word = ((wl >> 16) & 0xFFFF) | (wh & -65536)
                tab_v[pl.ds(off + s, _L)] = plsc.bitcast(word, jnp.float32)

            return carry

        lax.fori_loop(0, _PCHUNK, pack_chunk, 0)

    pack_table(a_hbm, ta_v)
    pack_table(b_hbm, tb_v)

    def start_in(c, b):
        off = base + c * _C
        pltpu.async_copy(iu_hbm.at[pl.ds(off, _C)], iu_v[b], insem[b])
        pltpu.async_copy(iv_hbm.at[pl.ds(off, _C)], iv_v[b], insem[b])
        pltpu.async_copy(d_hbm.at[pl.ds(off, _C)], d_v[b], insem[b])

    def wait_in(b):
        pltpu.make_async_copy(iu_hbm.at[pl.ds(0, _C)], iu_v[b],
                              insem[b]).wait()
        pltpu.make_async_copy(iv_hbm.at[pl.ds(0, _C)], iv_v[b],
                              insem[b]).wait()
        pltpu.make_async_copy(d_hbm.at[pl.ds(0, _C)], d_v[b],
                              insem[b]).wait()

    def wait_out(b):
        pltpu.make_async_copy(o_v[b], out_hbm.at[pl.ds(0, _C)],
                              outsem[b]).wait()

    start_in(0, 0)

    def pair_body(p, carry):
        for b in range(2):
            c = 2 * p + b
            if b == 0:
                start_in(c + 1, 1)
            else:
                @pl.when(p < _NCHUNK // 2 - 1)
                def _():
                    start_in(c + 1, 0)
            wait_in(b)

            @pl.when(c >= 2)
            def _():
                wait_out(b)

            ta, tb, iub, ivb, db, ob = (
                ta_v, tb_v, iu_v[b], iv_v[b], d_v[b], o_v[b])

            @plsc.parallel_loop(0, _VPC, 1, unroll=5)
            def _(k):
                s = k * _L
                iu = iub[pl.ds(s, _L)]
                iv = ivb[pl.ds(s, _L)]
                gu = iu >= _H
                gv = iv >= _H
                ju = jnp.where(gu, iu - _H, iu)
                jv = jnp.where(gv, iv - _H, iv)
                wu = plsc.bitcast(plsc.load_gather(ta, [ju]), jnp.int32)
                wv = plsc.bitcast(plsc.load_gather(tb, [jv]), jnp.int32)
                qu = plsc.bitcast(
                    jnp.where(gu, wu & -65536, wu << 16), jnp.float32)
                qv = plsc.bitcast(
                    jnp.where(gv, wv & -65536, wv << 16), jnp.float32)
                dd = db[pl.ds(s, _L)]
                ob[pl.ds(s, _L)] = (KE * qu) * qv / dd

            off = base + c * _C
            pltpu.async_copy(o_v[b], out_hbm.at[pl.ds(off, _C)], outsem[b])
        return carry

    lax.fori_loop(0, _NCHUNK // 2, pair_body, 0)
    wait_out(0)
    wait_out(1)


def kernel(mlmm_distances_uv, atomic_charges, mlmm_atomic_charges,
           mlmm_idx_u, mlmm_idx_v):
    mesh = plsc.VectorSubcoreMesh(core_axis_name="c", subcore_axis_name="s")
    run = pl.kernel(
        _body,
        out_type=jax.ShapeDtypeStruct((_E,), jnp.float32),
        mesh=mesh,
        compiler_params=pltpu.CompilerParams(needs_layout_passes=False),
        scratch_types=[
            pltpu.VMEM((_H,), jnp.float32),
            pltpu.VMEM((_H,), jnp.float32),
            pltpu.VMEM((_C,), jnp.int32),
            pltpu.VMEM((_C,), jnp.int32),
            pltpu.VMEM((_C,), jnp.int32),
            pltpu.VMEM((_C,), jnp.int32),
            pltpu.VMEM((_C,), jnp.float32),
            pltpu.VMEM((_C,), jnp.float32),
            pltpu.VMEM((_C,), jnp.float32),
            pltpu.VMEM((_C,), jnp.float32),
            pltpu.SemaphoreType.DMA,
            pltpu.SemaphoreType.DMA,
            pltpu.SemaphoreType.DMA,
            pltpu.SemaphoreType.DMA,
        ],
    )
    return run(mlmm_distances_uv, atomic_charges, mlmm_atomic_charges,
               mlmm_idx_u, mlmm_idx_v)


# speedup vs baseline: 2.0675x; 1.3741x over previous
import jax
import jax.numpy as jnp
from jax import lax
from jax.experimental import pallas as pl
from jax.experimental.pallas import tpu as pltpu
from jax.experimental.pallas import tpu_sc as plsc

KE = 332.0637

_NC = 2
_NS = 16
_NW = _NC * _NS
_L = 16

_E = 6400000
_T = _E // _NW
_C = 2000
_NCHUNK = _T // _C
_VPC = _C // _L

_N_TAB = 100000
_H = _N_TAB // 2
_PCHUNK = _H // _C


def _body(d_hbm, a_hbm, b_hbm, iu_hbm, iv_hbm, out_hbm,
          ta_v, tb_v, iu0, iu1, iv0, iv1, d0, d1, o0, o1,
          s0l, s0h, s1l, s1h,
          insem0, insem1, outsem0, outsem1):
    wid = lax.axis_index("s") * _NC + lax.axis_index("c")
    base = wid * _T

    iu_v = (iu0, iu1)
    iv_v = (iv0, iv1)
    d_v = (d0, d1)
    o_v = (o0, o1)
    insem = (insem0, insem1)
    outsem = (outsem0, outsem1)

    def start_in(c, b):
        off = base + c * _C
        pltpu.async_copy(iu_hbm.at[pl.ds(off, _C)], iu_v[b], insem[b])
        pltpu.async_copy(iv_hbm.at[pl.ds(off, _C)], iv_v[b], insem[b])
        pltpu.async_copy(d_hbm.at[pl.ds(off, _C)], d_v[b], insem[b])

    start_in(0, 0)

    def pack_table(t_hbm, tab_v):
        def start(c, lo, hi, sem):
            off = c * _C
            pltpu.async_copy(t_hbm.at[pl.ds(off, _C)], lo, sem)
            pltpu.async_copy(t_hbm.at[pl.ds(_H + off, _C)], hi, sem)

        def wait(lo, hi, sem):
            pltpu.make_async_copy(t_hbm.at[pl.ds(0, _C)], lo, sem).wait()
            pltpu.make_async_copy(t_hbm.at[pl.ds(0, _C)], hi, sem).wait()

        def pack(c, lo, hi):
            @plsc.parallel_loop(0, _VPC, 1, unroll=5)
            def _(k):
                s = k * _L
                wl = plsc.bitcast(lo[pl.ds(s, _L)], jnp.int32) + 0x8000
                wh = plsc.bitcast(hi[pl.ds(s, _L)], jnp.int32) + 0x8000
                word = ((wl >> 16) & 0xFFFF) | (wh & -65536)
                tab_v[pl.ds(c * _C + s, _L)] = plsc.bitcast(word, jnp.float32)

        start(0, s0l, s0h, outsem0)

        def pair(p, carry):
            c0 = 2 * p
            start(c0 + 1, s1l, s1h, outsem1)
            wait(s0l, s0h, outsem0)
            pack(c0, s0l, s0h)
            start(c0 + 2, s0l, s0h, outsem0)
            wait(s1l, s1h, outsem1)
            pack(c0 + 1, s1l, s1h)
            return carry

        lax.fori_loop(0, (_PCHUNK - 1) // 2, pair, 0)
        wait(s0l, s0h, outsem0)
        pack(_PCHUNK - 1, s0l, s0h)

    pack_table(a_hbm, ta_v)
    pack_table(b_hbm, tb_v)

    def wait_in(b):
        pltpu.make_async_copy(iu_hbm.at[pl.ds(0, _C)], iu_v[b],
                              insem[b]).wait()
        pltpu.make_async_copy(iv_hbm.at[pl.ds(0, _C)], iv_v[b],
                              insem[b]).wait()
        pltpu.make_async_copy(d_hbm.at[pl.ds(0, _C)], d_v[b],
                              insem[b]).wait()

    def wait_out(b):
        pltpu.make_async_copy(o_v[b], out_hbm.at[pl.ds(0, _C)],
                              outsem[b]).wait()

    def pair_body(p, carry):
        for b in range(2):
            c = 2 * p + b
            if b == 0:
                start_in(c + 1, 1)
            else:
                @pl.when(p < _NCHUNK // 2 - 1)
                def _():
                    start_in(c + 1, 0)
            wait_in(b)

            @pl.when(c >= 2)
            def _():
                wait_out(b)

            ta, tb, iub, ivb, db, ob = (
                ta_v, tb_v, iu_v[b], iv_v[b], d_v[b], o_v[b])

            @plsc.parallel_loop(0, _VPC, 1, unroll=5)
            def _(k):
                s = k * _L
                iu = iub[pl.ds(s, _L)]
                iv = ivb[pl.ds(s, _L)]
                gu = iu >= _H
                gv = iv >= _H
                ju = jnp.where(gu, iu - _H, iu)
                jv = jnp.where(gv, iv - _H, iv)
                wu = plsc.bitcast(plsc.load_gather(ta, [ju]), jnp.int32)
                wv = plsc.bitcast(plsc.load_gather(tb, [jv]), jnp.int32)
                qu = plsc.bitcast(
                    jnp.where(gu, wu & -65536, wu << 16), jnp.float32)
                qv = plsc.bitcast(
                    jnp.where(gv, wv & -65536, wv << 16), jnp.float32)
                dd = db[pl.ds(s, _L)]
                ob[pl.ds(s, _L)] = (KE * qu) * qv / dd

            off = base + c * _C
            pltpu.async_copy(o_v[b], out_hbm.at[pl.ds(off, _C)], outsem[b])
        return carry

    lax.fori_loop(0, _NCHUNK // 2, pair_body, 0)
    wait_out(0)
    wait_out(1)


def kernel(mlmm_distances_uv, atomic_charges, mlmm_atomic_charges,
           mlmm_idx_u, mlmm_idx_v):
    mesh = plsc.VectorSubcoreMesh(core_axis_name="c", subcore_axis_name="s")
    run = pl.kernel(
        _body,
        out_type=jax.ShapeDtypeStruct((_E,), jnp.float32),
        mesh=mesh,
        compiler_params=pltpu.CompilerParams(needs_layout_passes=False),
        scratch_types=[
            pltpu.VMEM((_H,), jnp.float32),
            pltpu.VMEM((_H,), jnp.float32),
            pltpu.VMEM((_C,), jnp.int32),
            pltpu.VMEM((_C,), jnp.int32),
            pltpu.VMEM((_C,), jnp.int32),
            pltpu.VMEM((_C,), jnp.int32),
            pltpu.VMEM((_C,), jnp.float32),
            pltpu.VMEM((_C,), jnp.float32),
            pltpu.VMEM((_C,), jnp.float32),
            pltpu.VMEM((_C,), jnp.float32),
            pltpu.VMEM((_C,), jnp.float32),
            pltpu.VMEM((_C,), jnp.float32),
            pltpu.VMEM((_C,), jnp.float32),
            pltpu.VMEM((_C,), jnp.float32),
            pltpu.SemaphoreType.DMA,
            pltpu.SemaphoreType.DMA,
            pltpu.SemaphoreType.DMA,
            pltpu.SemaphoreType.DMA,
        ],
    )
    return run(mlmm_distances_uv, atomic_charges, mlmm_atomic_charges,
               mlmm_idx_u, mlmm_idx_v)
